# baseline (speedup 1.0000x reference)
import functools

import jax
import jax.numpy as jnp
from jax import lax
from jax.experimental import pallas as pl
from jax.experimental.pallas import tpu as pltpu
from jax.experimental.pallas import tpu_sc as plsc

N = 10000
E = 320000
R = 16
NBASES = 4
D = 128
G = R + 1

BN = 1000
NBLK = N // BN

CH = 128
EPAD = 327680
NCHUNK = EPAD // CH
WIN = 40
NW0 = NCHUNK // (16 * WIN)
TRASH = N

NPAD = 10240
ROWS_PER_TILE = NPAD // 16
DUMP = 128


def _wcat_body(comp_ref, bases_ref, wself_ref, out_ref):
    ws = []
    for r in range(R):
        w = comp_ref[r, 0] * bases_ref[0]
        for b in range(1, NBASES):
            w = w + comp_ref[r, b] * bases_ref[b]
        ws.append(w)
    ws.append(wself_ref[...])
    out_ref[...] = jnp.concatenate(ws, axis=1).astype(jnp.bfloat16)


def _wcat_call(comp, bases, wself):
    return pl.pallas_call(
        _wcat_body,
        in_specs=[
            pl.BlockSpec(memory_space=pltpu.SMEM),
            pl.BlockSpec((NBASES, D, D), lambda: (0, 0, 0)),
            pl.BlockSpec((D, D), lambda: (0, 0)),
        ],
        out_specs=pl.BlockSpec((D, G * D), lambda: (0, 0)),
        out_shape=jax.ShapeDtypeStruct((D, G * D), jnp.bfloat16),
    )(comp, bases, wself)


def _hw_body(wcat_ref, h_ref, out_ref):
    out_ref[...] = jnp.dot(h_ref[...].astype(jnp.bfloat16), wcat_ref[...],
                           preferred_element_type=jnp.float32)


def _hw_call(wcat, h):
    return pl.pallas_call(
        _hw_body,
        grid=(NBLK, G),
        in_specs=[
            pl.BlockSpec((D, D), lambda i, r: (0, r)),
            pl.BlockSpec((BN, D), lambda i, r: (i, 0)),
        ],
        out_specs=pl.BlockSpec((None, BN, D), lambda i, r: (r, i, 0)),
        out_shape=jax.ShapeDtypeStruct((G, N, D), jnp.float32),
    )(wcat, h)


def _gidx_body(src_ref, et_ref, out_ref):
    out_ref[...] = et_ref[...] * N + src_ref[...]


def _gidx_call(src2d, et2d):
    return pl.pallas_call(
        _gidx_body,
        grid=(NCHUNK // CH,),
        in_specs=[
            pl.BlockSpec((CH, CH), lambda i: (i, 0)),
            pl.BlockSpec((CH, CH), lambda i: (i, 0)),
        ],
        out_specs=pl.BlockSpec((CH, CH), lambda i: (i, 0)),
        out_shape=jax.ShapeDtypeStruct((NCHUNK, CH), jnp.int32),
    )(src2d, et2d)


def _sc_edges_body(gidx_h, dst_h, hw, out, gidx_v, dst2_v,
                   rows_a, rows_b, agg_sh, sem_a, sem_b):
    c = lax.axis_index("c")
    s = lax.axis_index("s")

    @pl.when(c == 0)
    def _():
        def _zero(i, carry):
            for k in range(D // 16):
                rows_a[i, pl.ds(k * 16, 16)] = jnp.zeros((16,), jnp.float32)
            return carry
        lax.fori_loop(0, DUMP, _zero, 0)
        for j in range(ROWS_PER_TILE // DUMP):
            pltpu.sync_copy(
                rows_a, agg_sh.at[pl.ds(s * ROWS_PER_TILE + j * DUMP, DUMP)])
        plsc.subcore_barrier()

        def _gather(l, buf, sem):
            return pltpu.make_async_copy(hw.at[gidx_v.at[l]], buf, sem)

        for w in range(NW0):
            row0 = s * (NW0 * WIN) + w * WIN
            pltpu.sync_copy(gidx_h.at[pl.ds(row0, WIN)], gidx_v)
            pltpu.sync_copy(dst_h.at[pl.ds(row0, WIN)], dst2_v)

            _gather(0, rows_a, sem_a).start()
            _gather(1, rows_b, sem_b).start()

            def _pair(p, carry):
                l = 2 * p
                for b in range(2):
                    buf, sem = (rows_a, sem_a) if b == 0 else (rows_b, sem_b)
                    lb = l + b
                    _gather(lb, buf, sem).wait()
                    pltpu.sync_copy(buf, agg_sh.at[dst2_v.at[lb]], add=True)
                    @pl.when(lb + 2 < WIN)
                    def _():
                        _gather(lb + 2, buf, sem).start()
                return carry
            lax.fori_loop(0, WIN // 2, _pair, 0)
        plsc.subcore_barrier()

        for j in range(ROWS_PER_TILE // DUMP):
            row0 = s * ROWS_PER_TILE + j * DUMP
            pltpu.sync_copy(agg_sh.at[pl.ds(row0, DUMP)], rows_a)
            pltpu.sync_copy(rows_a, out.at[pl.ds(row0, DUMP)])


def _sc_edges_call(gidx2d, dst2d, hw_flat):
    mesh = plsc.VectorSubcoreMesh(core_axis_name="c", subcore_axis_name="s")
    f = functools.partial(
        pl.kernel,
        out_type=jax.ShapeDtypeStruct((NPAD, D), jnp.float32),
        mesh=mesh,
        scratch_types=[
            pltpu.VMEM((WIN, CH), jnp.int32),
            pltpu.VMEM((WIN, CH), jnp.int32),
            pltpu.VMEM((CH, D), jnp.float32),
            pltpu.VMEM((CH, D), jnp.float32),
            pltpu.VMEM_SHARED((NPAD, D), jnp.float32),
            pltpu.SemaphoreType.DMA,
            pltpu.SemaphoreType.DMA,
        ],
    )(_sc_edges_body)
    return f(gidx2d, dst2d, hw_flat)


def _combine_body(agg_ref, self_ref, bias_ref, gamma_ref, beta_ref, out_ref,
                  *, act):
    x = agg_ref[...] + self_ref[...] + bias_ref[...]
    mu = jnp.mean(x, axis=-1, keepdims=True)
    xc = x - mu
    var = jnp.mean(xc * xc, axis=-1, keepdims=True)
    y = gamma_ref[...] * (xc * lax.rsqrt(var + 1e-5)) + beta_ref[...]
    if act:
        y = jnp.maximum(y, 0.0)
    out_ref[...] = y


def _combine_call(agg, hw2, bias, gamma, beta, act):
    return pl.pallas_call(
        functools.partial(_combine_body, act=act),
        grid=(NBLK,),
        in_specs=[
            pl.BlockSpec((BN, D), lambda i: (i, 0)),
            pl.BlockSpec((None, BN, D), lambda i: (R, i, 0)),
            pl.BlockSpec((1, D), lambda i: (0, 0)),
            pl.BlockSpec((1, D), lambda i: (0, 0)),
            pl.BlockSpec((1, D), lambda i: (0, 0)),
        ],
        out_specs=pl.BlockSpec((BN, D), lambda i: (i, 0)),
        out_shape=jax.ShapeDtypeStruct((N, D), jnp.float32),
    )(agg, hw2, bias.reshape(1, D), gamma.reshape(1, D), beta.reshape(1, D))


GB = 320
GC = 64


def _sc_gather_body(nodes, h2, out, idx_v, rows_v, sem):
    c = lax.axis_index("c")
    s = lax.axis_index("s")
    wid = c * 16 + s
    base = jnp.minimum(wid * GB, N - GB)
    for j in range(GB // GC):
        pltpu.sync_copy(nodes.at[pl.ds(base + j * GC, GC)], idx_v)
        pltpu.async_copy(h2.at[idx_v], rows_v, sem).wait()
        pltpu.sync_copy(rows_v, out.at[pl.ds(base + j * GC, GC)])


def _sc_gather_call(nodes, h2):
    mesh = plsc.VectorSubcoreMesh(core_axis_name="c", subcore_axis_name="s")
    f = functools.partial(
        pl.kernel,
        out_type=jax.ShapeDtypeStruct((N, D), jnp.float32),
        mesh=mesh,
        scratch_types=[
            pltpu.VMEM((GC,), jnp.int32),
            pltpu.VMEM((GC, D), jnp.float32),
            pltpu.SemaphoreType.DMA,
        ],
    )(_sc_gather_body)
    return f(nodes, h2)


def _layer(h, gidx2d, dst2d, bases, comp, wself, bias, gamma, beta, act):
    wcat = _wcat_call(comp, bases, wself)
    hw = _hw_call(wcat, h)
    agg = _sc_edges_call(gidx2d, dst2d, hw.reshape(G * N, D))
    return _combine_call(agg, hw, bias, gamma, beta, act)


def kernel(nodes, edge_index, etypes, node_feat, bases0, comp0, wself0,
           bias0, gamma0, beta0, bases1, comp1, wself1, bias1, gamma1,
           beta1):
    pad = EPAD - E
    src2d = jnp.concatenate(
        [edge_index[0], jnp.zeros((pad,), jnp.int32)]).reshape(NCHUNK, CH)
    et2d = jnp.concatenate(
        [etypes, jnp.zeros((pad,), jnp.int32)]).reshape(NCHUNK, CH)
    trash = TRASH + jnp.arange(pad, dtype=jnp.int32) % (NPAD - N)
    dst2d = jnp.concatenate([edge_index[1], trash]).reshape(NCHUNK, CH)
    gidx2d = _gidx_call(src2d, et2d)

    h1 = _layer(node_feat, gidx2d, dst2d, bases0, comp0, wself0,
                bias0, gamma0, beta0, True)
    h2 = _layer(h1, gidx2d, dst2d, bases1, comp1, wself1,
                bias1, gamma1, beta1, False)
    return _sc_gather_call(nodes, h2)


# speedup vs baseline: 1.6959x; 1.6959x over previous
import functools

import jax
import jax.numpy as jnp
from jax import lax
from jax.experimental import pallas as pl
from jax.experimental.pallas import tpu as pltpu
from jax.experimental.pallas import tpu_sc as plsc

N = 10000
E = 320000
R = 16
NBASES = 4
D = 128
G = R + 1

BN = 1000
NBLK = N // BN

CH = 128
NCHUNK = E // CH
NWORK = 32

NPAD = 10240
ROWS_PER_TILE = NPAD // 16
DUMP = 128


def _wcat_body(comp_ref, bases_ref, wself_ref, out_ref):
    ws = []
    for r in range(R):
        w = comp_ref[r, 0] * bases_ref[0]
        for b in range(1, NBASES):
            w = w + comp_ref[r, b] * bases_ref[b]
        ws.append(w)
    ws.append(wself_ref[...])
    out_ref[...] = jnp.concatenate(ws, axis=1).astype(jnp.bfloat16)


def _wcat_call(comp, bases, wself):
    return pl.pallas_call(
        _wcat_body,
        in_specs=[
            pl.BlockSpec(memory_space=pltpu.SMEM),
            pl.BlockSpec((NBASES, D, D), lambda: (0, 0, 0)),
            pl.BlockSpec((D, D), lambda: (0, 0)),
        ],
        out_specs=pl.BlockSpec((D, G * D), lambda: (0, 0)),
        out_shape=jax.ShapeDtypeStruct((D, G * D), jnp.bfloat16),
    )(comp, bases, wself)


def _hw_body(wcat_ref, h_ref, out_ref):
    out_ref[...] = jnp.dot(h_ref[...].astype(jnp.bfloat16), wcat_ref[...],
                           preferred_element_type=jnp.float32)


def _hw_call(wcat, h):
    return pl.pallas_call(
        _hw_body,
        grid=(NBLK, G),
        in_specs=[
            pl.BlockSpec((D, D), lambda i, r: (0, r)),
            pl.BlockSpec((BN, D), lambda i, r: (i, 0)),
        ],
        out_specs=pl.BlockSpec((None, BN, D), lambda i, r: (r, i, 0)),
        out_shape=jax.ShapeDtypeStruct((G, N, D), jnp.float32),
    )(wcat, h)


def _sc_edges_body(ei, et, hw, out, src_v, et_v, gidx_v, dst_v, rows_v,
                   stage_v, agg_sh, sem):
    c = lax.axis_index("c")
    s = lax.axis_index("s")
    wid = c * 16 + s

    def _zero(i, carry):
        for k in range(D // 16):
            stage_v[i, pl.ds(k * 16, 16)] = jnp.zeros((16,), jnp.float32)
        return carry
    lax.fori_loop(0, DUMP, _zero, 0)
    for j in range(ROWS_PER_TILE // DUMP):
        pltpu.sync_copy(stage_v,
                        agg_sh.at[pl.ds(s * ROWS_PER_TILE + j * DUMP, DUMP)])
    plsc.subcore_barrier()

    start = wid * NCHUNK // NWORK
    stop = (wid + 1) * NCHUNK // NWORK

    def _chunk(ci, carry):
        off = ci * CH
        pltpu.sync_copy(ei.at[0, pl.ds(off, CH)], src_v)
        pltpu.sync_copy(ei.at[1, pl.ds(off, CH)], dst_v)
        pltpu.sync_copy(et.at[pl.ds(off, CH)], et_v)
        for i in range(CH // 16):
            sl = pl.ds(i * 16, 16)
            gidx_v[sl] = et_v[sl] * N + src_v[sl]
        pltpu.async_copy(hw.at[gidx_v], rows_v, sem).wait()
        pltpu.sync_copy(rows_v, agg_sh.at[dst_v], add=True)
        return carry
    lax.fori_loop(start, stop, _chunk, 0)
    plsc.subcore_barrier()

    for j in range(ROWS_PER_TILE // DUMP):
        row0 = s * ROWS_PER_TILE + j * DUMP
        pltpu.sync_copy(agg_sh.at[pl.ds(row0, DUMP)], stage_v)
        pltpu.sync_copy(stage_v, out.at[c, pl.ds(row0, DUMP)])


def _sc_edges_call(edge_index, etypes, hw_flat):
    mesh = plsc.VectorSubcoreMesh(core_axis_name="c", subcore_axis_name="s")
    f = functools.partial(
        pl.kernel,
        out_type=jax.ShapeDtypeStruct((2, NPAD, D), jnp.float32),
        mesh=mesh,
        scratch_types=[
            pltpu.VMEM((CH,), jnp.int32),
            pltpu.VMEM((CH,), jnp.int32),
            pltpu.VMEM((CH,), jnp.int32),
            pltpu.VMEM((CH,), jnp.int32),
            pltpu.VMEM((CH, D), jnp.float32),
            pltpu.VMEM((DUMP, D), jnp.float32),
            pltpu.VMEM_SHARED((NPAD, D), jnp.float32),
            pltpu.SemaphoreType.DMA,
        ],
    )(_sc_edges_body)
    return f(edge_index, etypes, hw_flat)


def _combine_body(agg_ref, self_ref, bias_ref, gamma_ref, beta_ref, out_ref,
                  *, act):
    x = agg_ref[0] + agg_ref[1] + self_ref[...] + bias_ref[...]
    mu = jnp.mean(x, axis=-1, keepdims=True)
    xc = x - mu
    var = jnp.mean(xc * xc, axis=-1, keepdims=True)
    y = gamma_ref[...] * (xc * lax.rsqrt(var + 1e-5)) + beta_ref[...]
    if act:
        y = jnp.maximum(y, 0.0)
    out_ref[...] = y


def _combine_call(agg, hw, bias, gamma, beta, act):
    return pl.pallas_call(
        functools.partial(_combine_body, act=act),
        grid=(NBLK,),
        in_specs=[
            pl.BlockSpec((2, BN, D), lambda i: (0, i, 0)),
            pl.BlockSpec((None, BN, D), lambda i: (R, i, 0)),
            pl.BlockSpec((1, D), lambda i: (0, 0)),
            pl.BlockSpec((1, D), lambda i: (0, 0)),
            pl.BlockSpec((1, D), lambda i: (0, 0)),
        ],
        out_specs=pl.BlockSpec((BN, D), lambda i: (i, 0)),
        out_shape=jax.ShapeDtypeStruct((N, D), jnp.float32),
    )(agg, hw, bias.reshape(1, D), gamma.reshape(1, D), beta.reshape(1, D))


GB = 320
GC = 64


def _sc_gather_body(nodes, h2, out, idx_v, rows_v, sem):
    c = lax.axis_index("c")
    s = lax.axis_index("s")
    wid = c * 16 + s
    base = jnp.minimum(wid * GB, N - GB)
    for j in range(GB // GC):
        pltpu.sync_copy(nodes.at[pl.ds(base + j * GC, GC)], idx_v)
        pltpu.async_copy(h2.at[idx_v], rows_v, sem).wait()
        pltpu.sync_copy(rows_v, out.at[pl.ds(base + j * GC, GC)])


def _sc_gather_call(nodes, h2):
    mesh = plsc.VectorSubcoreMesh(core_axis_name="c", subcore_axis_name="s")
    f = functools.partial(
        pl.kernel,
        out_type=jax.ShapeDtypeStruct((N, D), jnp.float32),
        mesh=mesh,
        scratch_types=[
            pltpu.VMEM((GC,), jnp.int32),
            pltpu.VMEM((GC, D), jnp.float32),
            pltpu.SemaphoreType.DMA,
        ],
    )(_sc_gather_body)
    return f(nodes, h2)


def _layer(h, edge_index, etypes, bases, comp, wself, bias, gamma, beta,
           act):
    wcat = _wcat_call(comp, bases, wself)
    hw = _hw_call(wcat, h)
    agg = _sc_edges_call(edge_index, etypes, hw.reshape(G * N, D))
    return _combine_call(agg, hw, bias, gamma, beta, act)


def kernel(nodes, edge_index, etypes, node_feat, bases0, comp0, wself0,
           bias0, gamma0, beta0, bases1, comp1, wself1, bias1, gamma1,
           beta1):
    h1 = _layer(node_feat, edge_index, etypes, bases0, comp0, wself0,
                bias0, gamma0, beta0, True)
    h2 = _layer(h1, edge_index, etypes, bases1, comp1, wself1,
                bias1, gamma1, beta1, False)
    return _sc_gather_call(nodes, h2)


# speedup vs baseline: 2.1990x; 1.2966x over previous
import functools

import jax
import jax.numpy as jnp
from jax import lax
from jax.experimental import pallas as pl
from jax.experimental.pallas import tpu as pltpu
from jax.experimental.pallas import tpu_sc as plsc

N = 10000
E = 320000
R = 16
NBASES = 4
D = 128
G = R + 1

BN = 1000
NBLK = N // BN

CH = 128
NCHUNK = E // CH
NWORK = 32

NPAD = 10240
ROWS_PER_TILE = NPAD // 16
DUMP = 128


def _wcat_body(comp_ref, bases_ref, wself_ref, out_ref):
    ws = []
    for r in range(R):
        w = comp_ref[r, 0] * bases_ref[0]
        for b in range(1, NBASES):
            w = w + comp_ref[r, b] * bases_ref[b]
        ws.append(w)
    ws.append(wself_ref[...])
    out_ref[...] = jnp.concatenate(ws, axis=1).astype(jnp.bfloat16)


def _wcat_call(comp, bases, wself):
    return pl.pallas_call(
        _wcat_body,
        in_specs=[
            pl.BlockSpec(memory_space=pltpu.SMEM),
            pl.BlockSpec((NBASES, D, D), lambda: (0, 0, 0)),
            pl.BlockSpec((D, D), lambda: (0, 0)),
        ],
        out_specs=pl.BlockSpec((D, G * D), lambda: (0, 0)),
        out_shape=jax.ShapeDtypeStruct((D, G * D), jnp.bfloat16),
    )(comp, bases, wself)


def _hw_body(wcat_ref, h_ref, out_ref):
    out_ref[...] = jnp.dot(h_ref[...].astype(jnp.bfloat16), wcat_ref[...],
                           preferred_element_type=jnp.float32)


def _hw_call(wcat, h):
    return pl.pallas_call(
        _hw_body,
        grid=(NBLK, G),
        in_specs=[
            pl.BlockSpec((D, D), lambda i, r: (0, r)),
            pl.BlockSpec((BN, D), lambda i, r: (i, 0)),
        ],
        out_specs=pl.BlockSpec((None, BN, D), lambda i, r: (r, i, 0)),
        out_shape=jax.ShapeDtypeStruct((G, N, D), jnp.float32),
    )(wcat, h)


def _sc_edges_body(ei, et, hw, out, src_v, et_v, gidx_v, dst_v, rows_v,
                   gidx_b, dst_b, rows_b, agg_sh, sem, sem_b):
    stage_v = rows_v
    c = lax.axis_index("c")
    s = lax.axis_index("s")
    wid = c * 16 + s

    def _zero(i, carry):
        for k in range(D // 16):
            stage_v[i, pl.ds(k * 16, 16)] = jnp.zeros((16,), jnp.float32)
        return carry
    lax.fori_loop(0, DUMP, _zero, 0)
    for j in range(ROWS_PER_TILE // DUMP):
        pltpu.sync_copy(stage_v,
                        agg_sh.at[pl.ds(s * ROWS_PER_TILE + j * DUMP, DUMP)])
    plsc.subcore_barrier()

    pstart = wid * (NCHUNK // 2) // NWORK
    pstop = (wid + 1) * (NCHUNK // 2) // NWORK

    def _load(ci, gv, dv):
        off = ci * CH
        pltpu.sync_copy(ei.at[0, pl.ds(off, CH)], src_v)
        pltpu.sync_copy(ei.at[1, pl.ds(off, CH)], dv)
        pltpu.sync_copy(et.at[pl.ds(off, CH)], et_v)
        for i in range(CH // 16):
            sl = pl.ds(i * 16, 16)
            gv[sl] = et_v[sl] * N + src_v[sl]

    _load(2 * pstart, gidx_v, dst_v)
    pltpu.async_copy(hw.at[gidx_v], rows_v, sem)

    def _pair(p, carry):
        l0 = 2 * p
        _load(l0 + 1, gidx_b, dst_b)
        pltpu.make_async_copy(hw.at[gidx_v], rows_v, sem).wait()
        pltpu.async_copy(hw.at[gidx_b], rows_b, sem_b)
        pltpu.sync_copy(rows_v, agg_sh.at[dst_v], add=True)
        more = p + 1 < pstop
        @pl.when(more)
        def _():
            _load(l0 + 2, gidx_v, dst_v)
        pltpu.make_async_copy(hw.at[gidx_b], rows_b, sem_b).wait()
        @pl.when(more)
        def _():
            pltpu.async_copy(hw.at[gidx_v], rows_v, sem)
        pltpu.sync_copy(rows_b, agg_sh.at[dst_b], add=True)
        return carry
    lax.fori_loop(pstart, pstop, _pair, 0)
    plsc.subcore_barrier()

    for j in range(ROWS_PER_TILE // DUMP):
        row0 = s * ROWS_PER_TILE + j * DUMP
        pltpu.sync_copy(agg_sh.at[pl.ds(row0, DUMP)], stage_v)
        pltpu.sync_copy(stage_v, out.at[c, pl.ds(row0, DUMP)])


def _sc_edges_call(edge_index, etypes, hw_flat):
    mesh = plsc.VectorSubcoreMesh(core_axis_name="c", subcore_axis_name="s")
    f = functools.partial(
        pl.kernel,
        out_type=jax.ShapeDtypeStruct((2, NPAD, D), jnp.float32),
        mesh=mesh,
        scratch_types=[
            pltpu.VMEM((CH,), jnp.int32),
            pltpu.VMEM((CH,), jnp.int32),
            pltpu.VMEM((CH,), jnp.int32),
            pltpu.VMEM((CH,), jnp.int32),
            pltpu.VMEM((CH, D), jnp.float32),
            pltpu.VMEM((CH,), jnp.int32),
            pltpu.VMEM((CH,), jnp.int32),
            pltpu.VMEM((CH, D), jnp.float32),
            pltpu.VMEM_SHARED((NPAD, D), jnp.float32),
            pltpu.SemaphoreType.DMA,
            pltpu.SemaphoreType.DMA,
        ],
    )(_sc_edges_body)
    return f(edge_index, etypes, hw_flat)


def _combine_body(agg_ref, self_ref, bias_ref, gamma_ref, beta_ref, out_ref,
                  *, act):
    x = agg_ref[0] + agg_ref[1] + self_ref[...] + bias_ref[...]
    mu = jnp.mean(x, axis=-1, keepdims=True)
    xc = x - mu
    var = jnp.mean(xc * xc, axis=-1, keepdims=True)
    y = gamma_ref[...] * (xc * lax.rsqrt(var + 1e-5)) + beta_ref[...]
    if act:
        y = jnp.maximum(y, 0.0)
    out_ref[...] = y


def _combine_call(agg, hw, bias, gamma, beta, act):
    return pl.pallas_call(
        functools.partial(_combine_body, act=act),
        grid=(NBLK,),
        in_specs=[
            pl.BlockSpec((2, BN, D), lambda i: (0, i, 0)),
            pl.BlockSpec((None, BN, D), lambda i: (R, i, 0)),
            pl.BlockSpec((1, D), lambda i: (0, 0)),
            pl.BlockSpec((1, D), lambda i: (0, 0)),
            pl.BlockSpec((1, D), lambda i: (0, 0)),
        ],
        out_specs=pl.BlockSpec((BN, D), lambda i: (i, 0)),
        out_shape=jax.ShapeDtypeStruct((N, D), jnp.float32),
    )(agg, hw, bias.reshape(1, D), gamma.reshape(1, D), beta.reshape(1, D))


GB = 320
GC = 64


def _sc_gather_body(nodes, h2, out, idx_v, rows_v, sem):
    c = lax.axis_index("c")
    s = lax.axis_index("s")
    wid = c * 16 + s
    base = jnp.minimum(wid * GB, N - GB)
    for j in range(GB // GC):
        pltpu.sync_copy(nodes.at[pl.ds(base + j * GC, GC)], idx_v)
        pltpu.async_copy(h2.at[idx_v], rows_v, sem).wait()
        pltpu.sync_copy(rows_v, out.at[pl.ds(base + j * GC, GC)])


def _sc_gather_call(nodes, h2):
    mesh = plsc.VectorSubcoreMesh(core_axis_name="c", subcore_axis_name="s")
    f = functools.partial(
        pl.kernel,
        out_type=jax.ShapeDtypeStruct((N, D), jnp.float32),
        mesh=mesh,
        scratch_types=[
            pltpu.VMEM((GC,), jnp.int32),
            pltpu.VMEM((GC, D), jnp.float32),
            pltpu.SemaphoreType.DMA,
        ],
    )(_sc_gather_body)
    return f(nodes, h2)


def _layer(h, edge_index, etypes, bases, comp, wself, bias, gamma, beta,
           act):
    wcat = _wcat_call(comp, bases, wself)
    hw = _hw_call(wcat, h)
    agg = _sc_edges_call(edge_index, etypes, hw.reshape(G * N, D))
    return _combine_call(agg, hw, bias, gamma, beta, act)


def kernel(nodes, edge_index, etypes, node_feat, bases0, comp0, wself0,
           bias0, gamma0, beta0, bases1, comp1, wself1, bias1, gamma1,
           beta1):
    h1 = _layer(node_feat, edge_index, etypes, bases0, comp0, wself0,
                bias0, gamma0, beta0, True)
    h2 = _layer(h1, edge_index, etypes, bases1, comp1, wself1,
                bias1, gamma1, beta1, False)
    return _sc_gather_call(nodes, h2)


# speedup vs baseline: 2.9200x; 1.3279x over previous
import functools

import jax
import jax.numpy as jnp
from jax import lax
from jax.experimental import pallas as pl
from jax.experimental.pallas import tpu as pltpu
from jax.experimental.pallas import tpu_sc as plsc

N = 10000
E = 320000
R = 16
NBASES = 4
D = 128
G = R + 1

BN = 1000
NBLK = N // BN

CH = 128
NCHUNK = E // CH
NWORK = 32

NPAD = 10240
ROWS_PER_TILE = NPAD // 16
DUMP = 128


def _wcat_body(comp_ref, bases_ref, wself_ref, out_ref):
    ws = []
    for r in range(R):
        w = comp_ref[r, 0] * bases_ref[0]
        for b in range(1, NBASES):
            w = w + comp_ref[r, b] * bases_ref[b]
        ws.append(w)
    ws.append(wself_ref[...])
    out_ref[...] = jnp.concatenate(ws, axis=1).astype(jnp.bfloat16)


def _wcat_call(comp, bases, wself):
    return pl.pallas_call(
        _wcat_body,
        in_specs=[
            pl.BlockSpec(memory_space=pltpu.SMEM),
            pl.BlockSpec((NBASES, D, D), lambda: (0, 0, 0)),
            pl.BlockSpec((D, D), lambda: (0, 0)),
        ],
        out_specs=pl.BlockSpec((D, G * D), lambda: (0, 0)),
        out_shape=jax.ShapeDtypeStruct((D, G * D), jnp.bfloat16),
    )(comp, bases, wself)


def _hw_body(wcat_ref, h_ref, out_ref, self_ref):
    y = jnp.dot(h_ref[...].astype(jnp.bfloat16), wcat_ref[...],
                preferred_element_type=jnp.float32)
    out_ref[...] = y.reshape(BN * G, D)
    self_ref[...] = y[:, R * D:]


def _hw_call(wcat, h):
    return pl.pallas_call(
        _hw_body,
        grid=(NBLK,),
        in_specs=[
            pl.BlockSpec((D, G * D), lambda i: (0, 0)),
            pl.BlockSpec((BN, D), lambda i: (i, 0)),
        ],
        out_specs=[
            pl.BlockSpec((BN * G, D), lambda i: (i, 0)),
            pl.BlockSpec((BN, D), lambda i: (i, 0)),
        ],
        out_shape=[
            jax.ShapeDtypeStruct((N * G, D), jnp.float32),
            jax.ShapeDtypeStruct((N, D), jnp.float32),
        ],
    )(wcat, h)


def _sc_edges_body(ei, et, hw, out, src_v, et_v, gidx_v, dst_v, rows_v,
                   gidx_b, dst_b, rows_b, agg_sh, sem, sem_b):
    stage_v = rows_v
    c = lax.axis_index("c")
    s = lax.axis_index("s")
    wid = c * 16 + s

    def _zero(i, carry):
        for k in range(D // 16):
            stage_v[i, pl.ds(k * 16, 16)] = jnp.zeros((16,), jnp.float32)
        return carry
    lax.fori_loop(0, DUMP, _zero, 0)
    for j in range(ROWS_PER_TILE // DUMP):
        pltpu.sync_copy(stage_v,
                        agg_sh.at[pl.ds(s * ROWS_PER_TILE + j * DUMP, DUMP)])
    plsc.subcore_barrier()

    pstart = wid * (NCHUNK // 2) // NWORK
    pstop = (wid + 1) * (NCHUNK // 2) // NWORK

    def _load(ci, gv, dv):
        off = ci * CH
        pltpu.sync_copy(ei.at[0, pl.ds(off, CH)], src_v)
        pltpu.sync_copy(ei.at[1, pl.ds(off, CH)], dv)
        pltpu.sync_copy(et.at[pl.ds(off, CH)], et_v)
        for i in range(CH // 16):
            sl = pl.ds(i * 16, 16)
            gv[sl] = src_v[sl] * G + et_v[sl]

    _load(2 * pstart, gidx_v, dst_v)
    pltpu.async_copy(hw.at[gidx_v], rows_v, sem)

    def _pair(p, carry):
        l0 = 2 * p
        _load(l0 + 1, gidx_b, dst_b)
        pltpu.make_async_copy(hw.at[gidx_v], rows_v, sem).wait()
        pltpu.async_copy(hw.at[gidx_b], rows_b, sem_b)
        pltpu.sync_copy(rows_v, agg_sh.at[dst_v], add=True)
        more = p + 1 < pstop
        @pl.when(more)
        def _():
            _load(l0 + 2, gidx_v, dst_v)
        pltpu.make_async_copy(hw.at[gidx_b], rows_b, sem_b).wait()
        @pl.when(more)
        def _():
            pltpu.async_copy(hw.at[gidx_v], rows_v, sem)
        pltpu.sync_copy(rows_b, agg_sh.at[dst_b], add=True)
        return carry
    lax.fori_loop(pstart, pstop, _pair, 0)
    plsc.subcore_barrier()

    for j in range(ROWS_PER_TILE // DUMP):
        row0 = s * ROWS_PER_TILE + j * DUMP
        pltpu.sync_copy(agg_sh.at[pl.ds(row0, DUMP)], stage_v)
        pltpu.sync_copy(stage_v, out.at[c, pl.ds(row0, DUMP)])


def _sc_edges_call(edge_index, etypes, hw_flat):
    mesh = plsc.VectorSubcoreMesh(core_axis_name="c", subcore_axis_name="s")
    f = functools.partial(
        pl.kernel,
        out_type=jax.ShapeDtypeStruct((2, NPAD, D), jnp.float32),
        mesh=mesh,
        scratch_types=[
            pltpu.VMEM((CH,), jnp.int32),
            pltpu.VMEM((CH,), jnp.int32),
            pltpu.VMEM((CH,), jnp.int32),
            pltpu.VMEM((CH,), jnp.int32),
            pltpu.VMEM((CH, D), jnp.float32),
            pltpu.VMEM((CH,), jnp.int32),
            pltpu.VMEM((CH,), jnp.int32),
            pltpu.VMEM((CH, D), jnp.float32),
            pltpu.VMEM_SHARED((NPAD, D), jnp.float32),
            pltpu.SemaphoreType.DMA,
            pltpu.SemaphoreType.DMA,
        ],
    )(_sc_edges_body)
    return f(edge_index, etypes, hw_flat)


def _combine_body(agg_ref, self_ref, bias_ref, gamma_ref, beta_ref, out_ref,
                  *, act):
    x = agg_ref[0] + agg_ref[1] + self_ref[...] + bias_ref[...]
    mu = jnp.mean(x, axis=-1, keepdims=True)
    xc = x - mu
    var = jnp.mean(xc * xc, axis=-1, keepdims=True)
    y = gamma_ref[...] * (xc * lax.rsqrt(var + 1e-5)) + beta_ref[...]
    if act:
        y = jnp.maximum(y, 0.0)
    out_ref[...] = y


def _combine_call(agg, selfp, bias, gamma, beta, act):
    return pl.pallas_call(
        functools.partial(_combine_body, act=act),
        grid=(NBLK,),
        in_specs=[
            pl.BlockSpec((2, BN, D), lambda i: (0, i, 0)),
            pl.BlockSpec((BN, D), lambda i: (i, 0)),
            pl.BlockSpec((1, D), lambda i: (0, 0)),
            pl.BlockSpec((1, D), lambda i: (0, 0)),
            pl.BlockSpec((1, D), lambda i: (0, 0)),
        ],
        out_specs=pl.BlockSpec((BN, D), lambda i: (i, 0)),
        out_shape=jax.ShapeDtypeStruct((N, D), jnp.float32),
    )(agg, selfp, bias.reshape(1, D), gamma.reshape(1, D), beta.reshape(1, D))


GB = 320
GC = 64


def _sc_gather_body(nodes, h2, out, idx_v, rows_v, sem):
    c = lax.axis_index("c")
    s = lax.axis_index("s")
    wid = c * 16 + s
    base = jnp.minimum(wid * GB, N - GB)
    for j in range(GB // GC):
        pltpu.sync_copy(nodes.at[pl.ds(base + j * GC, GC)], idx_v)
        pltpu.async_copy(h2.at[idx_v], rows_v, sem).wait()
        pltpu.sync_copy(rows_v, out.at[pl.ds(base + j * GC, GC)])


def _sc_gather_call(nodes, h2):
    mesh = plsc.VectorSubcoreMesh(core_axis_name="c", subcore_axis_name="s")
    f = functools.partial(
        pl.kernel,
        out_type=jax.ShapeDtypeStruct((N, D), jnp.float32),
        mesh=mesh,
        scratch_types=[
            pltpu.VMEM((GC,), jnp.int32),
            pltpu.VMEM((GC, D), jnp.float32),
            pltpu.SemaphoreType.DMA,
        ],
    )(_sc_gather_body)
    return f(nodes, h2)


def _layer(h, edge_index, etypes, bases, comp, wself, bias, gamma, beta,
           act):
    wcat = _wcat_call(comp, bases, wself)
    hwflat, selfp = _hw_call(wcat, h)
    agg = _sc_edges_call(edge_index, etypes, hwflat)
    return _combine_call(agg, selfp, bias, gamma, beta, act)


def kernel(nodes, edge_index, etypes, node_feat, bases0, comp0, wself0,
           bias0, gamma0, beta0, bases1, comp1, wself1, bias1, gamma1,
           beta1):
    h1 = _layer(node_feat, edge_index, etypes, bases0, comp0, wself0,
                bias0, gamma0, beta0, True)
    h2 = _layer(h1, edge_index, etypes, bases1, comp1, wself1,
                bias1, gamma1, beta1, False)
    return _sc_gather_call(nodes, h2)
